# baseline (speedup 1.0000x reference)
import jax
import jax.numpy as jnp
from jax.experimental import pallas as pl

_SLEN = 512
_PT = 16
_STEP = 8
_NB = _SLEN // _STEP
_NPD = (_SLEN - _PT) // _STEP + 1
_HID = 64
_ODIM = 32


def _enc_kernel(blk_ref, q_ref, b1_ref, w2_ref, b2_ref, out_ref):
    blk = blk_ref[0]
    p = jnp.dot(blk, q_ref[...], preferred_element_type=jnp.float32)
    x = p[:, 0:64] + jnp.roll(p[:, 128:192], -64, axis=0)
    y = p[:, 64:128] + jnp.roll(p[:, 192:256], -64, axis=0)
    h = jnp.maximum(x + jnp.roll(y, -1, axis=0) + b1_ref[...], 0.0)
    o = jnp.dot(h.astype(jnp.bfloat16), w2_ref[...],
                preferred_element_type=jnp.float32) + b2_ref[...]
    out_ref[0] = o.T


def kernel(image, W1, b1, W2, b2):
    B = image.shape[0]
    blk = (
        image.astype(jnp.bfloat16)
        .reshape(B, _NB, _STEP, _NB, _STEP)
        .transpose(0, 1, 3, 2, 4)
        .reshape(B, _NB * _NB, _STEP * _STEP)
    )
    w1r = W1.reshape(_PT, _PT, _HID)
    q = jnp.concatenate(
        [
            w1r[0:8, 0:8].reshape(64, _HID),
            w1r[0:8, 8:16].reshape(64, _HID),
            w1r[8:16, 0:8].reshape(64, _HID),
            w1r[8:16, 8:16].reshape(64, _HID),
        ],
        axis=1,
    ).astype(jnp.bfloat16)

    out = pl.pallas_call(
        _enc_kernel,
        grid=(B,),
        in_specs=[
            pl.BlockSpec((1, _NB * _NB, 64), lambda b: (b, 0, 0)),
            pl.BlockSpec((64, 4 * _HID), lambda b: (0, 0)),
            pl.BlockSpec((1, _HID), lambda b: (0, 0)),
            pl.BlockSpec((_HID, _ODIM), lambda b: (0, 0)),
            pl.BlockSpec((1, _ODIM), lambda b: (0, 0)),
        ],
        out_specs=pl.BlockSpec((1, _ODIM, _NB * _NB), lambda b: (b, 0, 0)),
        out_shape=jax.ShapeDtypeStruct((B, _ODIM, _NB * _NB), jnp.float32),
    )(blk, q, b1.reshape(1, _HID), W2.astype(jnp.bfloat16), b2.reshape(1, _ODIM))

    out = out.reshape(B, _ODIM, _NB, _NB)[:, :, :_NPD, :_NPD]
    return out.transpose(0, 2, 3, 1).reshape(B * _NPD * _NPD, _ODIM)


# speedup vs baseline: 1.0263x; 1.0263x over previous
import jax
import jax.numpy as jnp
from jax.experimental import pallas as pl
from jax.experimental.pallas import tpu as pltpu

_SLEN = 512
_PT = 16
_STEP = 8
_NB = _SLEN // _STEP
_NPD = (_SLEN - _PT) // _STEP + 1
_HID = 64
_ODIM = 32


def _enc_kernel(ph_ref, q_ref, b1_ref, w2_ref, b2_ref, out_ref, p_ref):
    def body(i, carry):
        st_i = ph_ref[0, i]
        p_ref[pl.ds(i * _NB, _NB), :] = jax.lax.dot_general(
            st_i, q_ref[...],
            dimension_numbers=(((0,), (0,)), ((), ())),
            preferred_element_type=jnp.float32,
        )
        return carry

    jax.lax.fori_loop(0, _NB, body, 0, unroll=8)

    p = p_ref[...]
    x = p[:, 0:64] + jnp.roll(p[:, 128:192], -64, axis=0)
    y = p[:, 64:128] + jnp.roll(p[:, 192:256], -64, axis=0)
    h = jnp.maximum(x + jnp.roll(y, -1, axis=0) + b1_ref[...], 0.0)
    o = jnp.dot(h.astype(jnp.bfloat16), w2_ref[...],
                preferred_element_type=jnp.float32) + b2_ref[...]
    out_ref[0] = o.T


def kernel(image, W1, b1, W2, b2):
    B = image.shape[0]
    ph = (
        image.astype(jnp.bfloat16)
        .reshape(B, _NB, _STEP, _NB, _STEP)
        .transpose(0, 1, 2, 4, 3)
        .reshape(B, _NB, _STEP * _STEP, _NB)
    )
    w1r = W1.reshape(_PT, _PT, _HID)
    q = jnp.concatenate(
        [
            w1r[0:8, 0:8].reshape(64, _HID),
            w1r[0:8, 8:16].reshape(64, _HID),
            w1r[8:16, 0:8].reshape(64, _HID),
            w1r[8:16, 8:16].reshape(64, _HID),
        ],
        axis=1,
    ).astype(jnp.bfloat16)

    out = pl.pallas_call(
        _enc_kernel,
        grid=(B,),
        in_specs=[
            pl.BlockSpec((1, _NB, _STEP * _STEP, _NB), lambda b: (b, 0, 0, 0)),
            pl.BlockSpec((64, 4 * _HID), lambda b: (0, 0)),
            pl.BlockSpec((1, _HID), lambda b: (0, 0)),
            pl.BlockSpec((_HID, _ODIM), lambda b: (0, 0)),
            pl.BlockSpec((1, _ODIM), lambda b: (0, 0)),
        ],
        out_specs=pl.BlockSpec((1, _ODIM, _NB * _NB), lambda b: (b, 0, 0)),
        out_shape=jax.ShapeDtypeStruct((B, _ODIM, _NB * _NB), jnp.float32),
        scratch_shapes=[pltpu.VMEM((_NB * _NB, 4 * _HID), jnp.float32)],
    )(ph, q, b1.reshape(1, _HID), W2.astype(jnp.bfloat16), b2.reshape(1, _ODIM))

    out = out.reshape(B, _ODIM, _NB, _NB)[:, :, :_NPD, :_NPD]
    return out.transpose(0, 2, 3, 1).reshape(B * _NPD * _NPD, _ODIM)
